# baseline (speedup 1.0000x reference)
import functools

import jax
import jax.numpy as jnp
from jax.experimental import pallas as pl

NUM_AGENTS = 32768
NUM_ABS = 64
EMB_DIM = 256
ACT_DIM = 1024
BM = 512


def _body(u_ref, al_ref, emb_ref, aa_ref, w1t_ref, w0_ref, b_ref, out_ref):
    u = u_ref[...]
    s = al_ref[...] - jnp.log(-jnp.log(u))
    m = jnp.max(s, axis=1, keepdims=True)
    iota = jax.lax.broadcasted_iota(jnp.int32, s.shape, 1)
    idx = jnp.min(jnp.where(s >= m, iota, NUM_ABS), axis=1, keepdims=True)
    assigned = jnp.sum(
        jnp.where(iota == idx, aa_ref[...], 0.0), axis=1, keepdims=True
    )

    acc = jnp.dot(emb_ref[...].astype(jnp.bfloat16),
                  w1t_ref[...].astype(jnp.bfloat16),
                  preferred_element_type=jnp.float32)
    logits = acc + assigned * w0_ref[...] + b_ref[...]

    mx = jnp.max(logits, axis=1, keepdims=True)
    e = jnp.exp(logits - mx)
    out_ref[...] = e * (1.0 / jnp.sum(e, axis=1, keepdims=True))


@jax.jit
def kernel(abs_actions, gumbel_u, assigner_logits, emb_table, W, b):
    w1t = W[:, 1:].T
    w0 = W[:, 0].reshape(1, ACT_DIM)
    br = b.reshape(1, ACT_DIM)
    aa = abs_actions.reshape(1, NUM_ABS)

    grid = (NUM_AGENTS // BM,)
    return pl.pallas_call(
        _body,
        grid=grid,
        in_specs=[
            pl.BlockSpec((BM, NUM_ABS), lambda i: (i, 0)),
            pl.BlockSpec((BM, NUM_ABS), lambda i: (i, 0)),
            pl.BlockSpec((BM, EMB_DIM), lambda i: (i, 0)),
            pl.BlockSpec((1, NUM_ABS), lambda i: (0, 0)),
            pl.BlockSpec((EMB_DIM, ACT_DIM), lambda i: (0, 0)),
            pl.BlockSpec((1, ACT_DIM), lambda i: (0, 0)),
            pl.BlockSpec((1, ACT_DIM), lambda i: (0, 0)),
        ],
        out_specs=pl.BlockSpec((BM, ACT_DIM), lambda i: (i, 0)),
        out_shape=jax.ShapeDtypeStruct((NUM_AGENTS, ACT_DIM), jnp.float32),
    )(gumbel_u, assigner_logits, emb_table, aa, w1t, w0, br)


# speedup vs baseline: 1.1921x; 1.1921x over previous
import functools

import jax
import jax.numpy as jnp
from jax.experimental import pallas as pl

NUM_AGENTS = 32768
NUM_ABS = 64
EMB_DIM = 256
ACT_DIM = 1024
BM = 1024


def _body(u_ref, al_ref, emb_ref, aa_ref, w1t_ref, w0_ref, b_ref, out_ref):
    u = u_ref[...]
    s = al_ref[...] - jnp.log(-jnp.log(u))
    m = jnp.max(s, axis=1, keepdims=True)
    iota = jax.lax.broadcasted_iota(jnp.int32, s.shape, 1)
    idx = jnp.min(jnp.where(s >= m, iota, NUM_ABS), axis=1, keepdims=True)
    assigned = jnp.sum(
        jnp.where(iota == idx, aa_ref[...], 0.0), axis=1, keepdims=True
    )

    acc = jnp.dot(emb_ref[...].astype(jnp.bfloat16),
                  w1t_ref[...].astype(jnp.bfloat16),
                  preferred_element_type=jnp.float32)
    logits = acc + assigned * w0_ref[...] + b_ref[...]

    mx = jnp.max(logits, axis=1, keepdims=True)
    e = jnp.exp(logits - mx)
    out_ref[...] = e * (1.0 / jnp.sum(e, axis=1, keepdims=True))


@jax.jit
def kernel(abs_actions, gumbel_u, assigner_logits, emb_table, W, b):
    w1t = W[:, 1:].T
    w0 = W[:, 0].reshape(1, ACT_DIM)
    br = b.reshape(1, ACT_DIM)
    aa = abs_actions.reshape(1, NUM_ABS)

    grid = (NUM_AGENTS // BM,)
    return pl.pallas_call(
        _body,
        grid=grid,
        in_specs=[
            pl.BlockSpec((BM, NUM_ABS), lambda i: (i, 0)),
            pl.BlockSpec((BM, NUM_ABS), lambda i: (i, 0)),
            pl.BlockSpec((BM, EMB_DIM), lambda i: (i, 0)),
            pl.BlockSpec((1, NUM_ABS), lambda i: (0, 0)),
            pl.BlockSpec((EMB_DIM, ACT_DIM), lambda i: (0, 0)),
            pl.BlockSpec((1, ACT_DIM), lambda i: (0, 0)),
            pl.BlockSpec((1, ACT_DIM), lambda i: (0, 0)),
        ],
        out_specs=pl.BlockSpec((BM, ACT_DIM), lambda i: (i, 0)),
        out_shape=jax.ShapeDtypeStruct((NUM_AGENTS, ACT_DIM), jnp.float32),
    )(gumbel_u, assigner_logits, emb_table, aa, w1t, w0, br)


# speedup vs baseline: 1.2914x; 1.0833x over previous
import functools

import jax
import jax.numpy as jnp
from jax.experimental import pallas as pl

NUM_AGENTS = 32768
NUM_ABS = 64
EMB_DIM = 256
ACT_DIM = 1024
BM = 2048


def _body(u_ref, al_ref, emb_ref, aa_ref, w1t_ref, w0_ref, b_ref, out_ref):
    u = u_ref[...]
    s = al_ref[...] - jnp.log(-jnp.log(u))
    m = jnp.max(s, axis=1, keepdims=True)
    iota = jax.lax.broadcasted_iota(jnp.int32, s.shape, 1)
    idx = jnp.min(jnp.where(s >= m, iota, NUM_ABS), axis=1, keepdims=True)
    assigned = jnp.sum(
        jnp.where(iota == idx, aa_ref[...], 0.0), axis=1, keepdims=True
    )

    acc = jnp.dot(emb_ref[...].astype(jnp.bfloat16),
                  w1t_ref[...].astype(jnp.bfloat16),
                  preferred_element_type=jnp.float32)
    logits = acc + assigned * w0_ref[...] + b_ref[...]

    mx = jnp.max(logits, axis=1, keepdims=True)
    e = jnp.exp(logits - mx)
    out_ref[...] = e * (1.0 / jnp.sum(e, axis=1, keepdims=True))


@jax.jit
def kernel(abs_actions, gumbel_u, assigner_logits, emb_table, W, b):
    w1t = W[:, 1:].T
    w0 = W[:, 0].reshape(1, ACT_DIM)
    br = b.reshape(1, ACT_DIM)
    aa = abs_actions.reshape(1, NUM_ABS)

    grid = (NUM_AGENTS // BM,)
    return pl.pallas_call(
        _body,
        grid=grid,
        in_specs=[
            pl.BlockSpec((BM, NUM_ABS), lambda i: (i, 0)),
            pl.BlockSpec((BM, NUM_ABS), lambda i: (i, 0)),
            pl.BlockSpec((BM, EMB_DIM), lambda i: (i, 0)),
            pl.BlockSpec((1, NUM_ABS), lambda i: (0, 0)),
            pl.BlockSpec((EMB_DIM, ACT_DIM), lambda i: (0, 0)),
            pl.BlockSpec((1, ACT_DIM), lambda i: (0, 0)),
            pl.BlockSpec((1, ACT_DIM), lambda i: (0, 0)),
        ],
        out_specs=pl.BlockSpec((BM, ACT_DIM), lambda i: (i, 0)),
        out_shape=jax.ShapeDtypeStruct((NUM_AGENTS, ACT_DIM), jnp.float32),
    )(gumbel_u, assigner_logits, emb_table, aa, w1t, w0, br)
